# Initial kernel scaffold; baseline (speedup 1.0000x reference)
import jax
import jax.numpy as jnp
from jax.experimental import pallas as pl


def kernel(node_features, node_idx, hedge_idx, W_he, b_he, W_node, b_node, ln_gamma, ln_beta):
    raise NotImplementedError("write your pallas kernel here")


# speedup vs baseline: 5.2429x; 5.2429x over previous
import functools

import jax
import jax.numpy as jnp
from jax import lax
from jax.experimental import pallas as pl
from jax.experimental.pallas import tpu as pltpu
from jax.experimental.pallas import tpu_sc as plsc

N_NODES = 10000
N_HYPEREDGES = 5000
N_INC = 320000
D = 128

NC = 2
NS = 16
NW = NC * NS
L = 16
PER_W = 10080
CHUNK = 80
NPASS = 2
NCHP = PER_W // (NPASS * CHUNK)

HP = 5120
NP = 10240


def _make_sc_stage(n_tab, n_out):
    rpt = n_out // NS
    mesh = plsc.VectorSubcoreMesh(
        core_axis_name="c", subcore_axis_name="s", num_cores=NC, num_subcores=NS
    )

    @functools.partial(
        pl.kernel,
        out_type=(
            jax.ShapeDtypeStruct((NC * n_out, D), jnp.float32),
            jax.ShapeDtypeStruct((NW, n_out), jnp.float32),
        ),
        mesh=mesh,
        scratch_types=[
            pltpu.VMEM((NCHP, CHUNK), jnp.int32),
            pltpu.VMEM((NCHP, CHUNK), jnp.int32),
            pltpu.VMEM((CHUNK, D), jnp.float32),
            pltpu.VMEM((n_out,), jnp.float32),
            pltpu.VMEM_SHARED((n_out, D), jnp.float32),
            pltpu.SemaphoreType.DMA,
        ],
        compiler_params=pltpu.CompilerParams(needs_layout_passes=False),
    )
    def sc_kernel(table, gidx4, sidx4, zrow, zcnt,
                  out_sum, out_cnt, gv, sv, rows, cnt_v, acc_sh, sem):
        cid = lax.axis_index("c")
        sid = lax.axis_index("s")
        wid = sid * NC + cid

        pltpu.sync_copy(zcnt, cnt_v)
        pltpu.sync_copy(zrow, acc_sh.at[pl.ds(sid * rpt, rpt)])
        plsc.subcore_barrier()

        ones16 = jnp.ones((L,), jnp.float32)

        def body(j, carry):
            pltpu.async_copy(table.at[gv.at[j]], rows, sem).wait()
            pltpu.sync_copy(rows, acc_sh.at[sv.at[j]], add=True)
            for k in range(CHUNK // L):
                idx16 = sv[j, pl.ds(k * L, L)]
                plsc.addupdate_scatter(cnt_v, [idx16], ones16)
            return carry

        for p in range(NPASS):
            pltpu.sync_copy(gidx4.at[wid, p], gv)
            pltpu.sync_copy(sidx4.at[wid, p], sv)
            lax.fori_loop(0, NCHP, body, 0)
        plsc.subcore_barrier()

        base = cid * n_out + sid * rpt
        pltpu.sync_copy(acc_sh.at[pl.ds(sid * rpt, rpt)], out_sum.at[pl.ds(base, rpt)])
        pltpu.sync_copy(cnt_v, out_cnt.at[wid])

    return sc_kernel


_sc_stage1 = _make_sc_stage(NP, HP)
_sc_stage2 = _make_sc_stage(HP, NP)


def _tc1_body(s0, s1, c, w, b, o):
    cnt = jnp.sum(c[...], axis=0)[:, None]
    m = (s0[...] + s1[...]) / jnp.maximum(cnt, 1.0)
    y = jnp.dot(m, w[...], preferred_element_type=jnp.float32) + b[...]
    o[...] = jnp.maximum(y, 0.0)


def _tc2_body(s0, s1, c, nf, w, b, g, be, o):
    cnt = jnp.sum(c[...], axis=0)[:, None]
    m = (s0[...] + s1[...]) / jnp.maximum(cnt, 1.0)
    x = jnp.dot(m, w[...], preferred_element_type=jnp.float32) + b[...] + nf[...]
    mu = jnp.mean(x, axis=-1, keepdims=True)
    var = jnp.mean((x - mu) ** 2, axis=-1, keepdims=True)
    x = (x - mu) * lax.rsqrt(var + 1e-5) * g[...] + be[...]
    o[...] = jnp.maximum(x, 0.0)


def _tc_stage1(parts, cnts, wT, b):
    B = 640
    nb = HP // B
    return pl.pallas_call(
        _tc1_body,
        grid=(nb,),
        in_specs=[
            pl.BlockSpec((B, D), lambda i: (i, 0)),
            pl.BlockSpec((B, D), lambda i: (i + nb, 0)),
            pl.BlockSpec((NW, B), lambda i: (0, i)),
            pl.BlockSpec((D, D), lambda i: (0, 0)),
            pl.BlockSpec((1, D), lambda i: (0, 0)),
        ],
        out_specs=pl.BlockSpec((B, D), lambda i: (i, 0)),
        out_shape=jax.ShapeDtypeStruct((HP, D), jnp.float32),
    )(parts, parts, cnts, wT, b)


def _tc_stage2(parts, cnts, nf_pad, wT, b, g, be):
    B = 640
    nb = NP // B
    return pl.pallas_call(
        _tc2_body,
        grid=(nb,),
        in_specs=[
            pl.BlockSpec((B, D), lambda i: (i, 0)),
            pl.BlockSpec((B, D), lambda i: (i + nb, 0)),
            pl.BlockSpec((NW, B), lambda i: (0, i)),
            pl.BlockSpec((B, D), lambda i: (i, 0)),
            pl.BlockSpec((D, D), lambda i: (0, 0)),
            pl.BlockSpec((1, D), lambda i: (0, 0)),
            pl.BlockSpec((1, D), lambda i: (0, 0)),
            pl.BlockSpec((1, D), lambda i: (0, 0)),
        ],
        out_specs=pl.BlockSpec((B, D), lambda i: (i, 0)),
        out_shape=jax.ShapeDtypeStruct((NP, D), jnp.float32),
    )(parts, parts, cnts, nf_pad, wT, b, g, be)


def kernel(node_features, node_idx, hedge_idx, W_he, b_he, W_node, b_node,
           ln_gamma, ln_beta):
    pad = PER_W - N_INC // NW
    nidx = node_idx.astype(jnp.int32).reshape(NW, N_INC // NW)
    nidx = jnp.pad(nidx, ((0, 0), (0, pad)), constant_values=NP - 1)
    nidx = nidx.reshape(NW, NPASS, NCHP, CHUNK)
    hidx = hedge_idx.astype(jnp.int32).reshape(NW, N_INC // NW)
    hidx = jnp.pad(hidx, ((0, 0), (0, pad)), constant_values=HP - 1)
    hidx = hidx.reshape(NW, NPASS, NCHP, CHUNK)

    nf_pad = jnp.pad(node_features, ((0, NP - N_NODES), (0, 0)))

    z1r = jnp.zeros((HP // NS, D), jnp.float32)
    z1c = jnp.zeros((HP,), jnp.float32)
    he_sum, he_cnt = _sc_stage1(nf_pad, nidx, hidx, z1r, z1c)
    he_feat = _tc_stage1(he_sum, he_cnt, W_he.T, b_he.reshape(1, D))

    z2r = jnp.zeros((NP // NS, D), jnp.float32)
    z2c = jnp.zeros((NP,), jnp.float32)
    nd_sum, nd_cnt = _sc_stage2(he_feat, hidx, nidx, z2r, z2c)

    out = _tc_stage2(nd_sum, nd_cnt, nf_pad, W_node.T, b_node.reshape(1, D),
                     ln_gamma.reshape(1, D), ln_beta.reshape(1, D))
    return out[:N_NODES]
